# Initial kernel scaffold; baseline (speedup 1.0000x reference)
import jax
import jax.numpy as jnp
from jax.experimental import pallas as pl


def kernel(x, edge_index, batch, W1, b1, W2, b2, W3, b3):
    raise NotImplementedError("write your pallas kernel here")


# speedup vs baseline: 26.4887x; 26.4887x over previous
import functools

import jax
import jax.numpy as jnp
from jax import lax
from jax.experimental import pallas as pl
from jax.experimental.pallas import tpu as pltpu
from jax.experimental.pallas import tpu_sc as plsc

NC = 2
NS = 16
NW = NC * NS
CH = 128
G = 64


def _pad_to(n: int, m: int) -> int:
    return -(-n // m) * m


@functools.lru_cache(maxsize=None)
def _make_deg(EP: int, NP: int):
    nchunks = EP // (NW * CH)
    rpt = NP // NS
    mesh = plsc.VectorSubcoreMesh(core_axis_name="c", subcore_axis_name="s")

    @functools.partial(
        pl.kernel,
        out_type=jax.ShapeDtypeStruct((NC * NP,), jnp.float32),
        mesh=mesh,
        scratch_types=[
            pltpu.VMEM_SHARED((NP,), jnp.float32),
            pltpu.VMEM((nchunks, CH), jnp.int32),
            pltpu.VMEM((CH,), jnp.float32),
        ],
    )
    def deg_kernel(dst2, zeros1, out, acc, dbuf, ones_v):
        c = lax.axis_index("c")
        s = lax.axis_index("s")
        tile = c * NS + s
        for i in range(CH // 16):
            ones_v[pl.ds(i * 16, 16)] = jnp.ones((16,), jnp.float32)
        pltpu.sync_copy(zeros1.at[pl.ds(s * rpt, rpt)], acc.at[pl.ds(s * rpt, rpt)])
        pltpu.sync_copy(dst2.at[pl.ds(tile * nchunks, nchunks)], dbuf)
        plsc.subcore_barrier()

        def body(j, carry):
            pltpu.sync_copy(ones_v, acc.at[dbuf.at[j]], add=True)
            return carry

        lax.fori_loop(0, nchunks, body, 0)
        plsc.subcore_barrier()
        pltpu.sync_copy(acc.at[pl.ds(s * rpt, rpt)],
                        out.at[pl.ds(c * NP + s * rpt, rpt)])

    return deg_kernel


@functools.lru_cache(maxsize=None)
def _make_mp(EP: int, NP: int, H: int):
    nchunks = EP // (NW * CH)
    rpt = NP // NS
    mesh = plsc.VectorSubcoreMesh(core_axis_name="c", subcore_axis_name="s")

    @functools.partial(
        pl.kernel,
        out_type=jax.ShapeDtypeStruct((NC, NP, H), jnp.float32),
        mesh=mesh,
        compiler_params=pltpu.CompilerParams(use_tc_tiling_on_sc=False),
        scratch_types=[
            pltpu.VMEM_SHARED((NP, H), jnp.float32),
            pltpu.VMEM((nchunks, CH), jnp.int32),
            pltpu.VMEM((nchunks, CH), jnp.int32),
            pltpu.VMEM((CH, H), jnp.float32),
            pltpu.VMEM((CH, H), jnp.float32),
            pltpu.SemaphoreType.DMA,
            pltpu.SemaphoreType.DMA,
        ],
    )
    def mp_kernel(table, src2, dst2, zeros2, out,
                  acc, sbuf, dbuf, rows0, rows1, sem0, sem1):
        c = lax.axis_index("c")
        s = lax.axis_index("s")
        tile = c * NS + s
        pltpu.sync_copy(zeros2.at[pl.ds(s * rpt, rpt)], acc.at[pl.ds(s * rpt, rpt)])
        pltpu.sync_copy(src2.at[pl.ds(tile * nchunks, nchunks)], sbuf)
        pltpu.sync_copy(dst2.at[pl.ds(tile * nchunks, nchunks)], dbuf)
        plsc.subcore_barrier()
        pltpu.async_copy(table.at[sbuf.at[0]], rows0, sem0)

        def body(jj, carry):
            j0 = 2 * jj
            j1 = j0 + 1
            pltpu.make_async_copy(table.at[pl.ds(0, CH)], rows0, sem0).wait()
            pltpu.async_copy(table.at[sbuf.at[j1]], rows1, sem1)
            pltpu.sync_copy(rows0, acc.at[dbuf.at[j0]], add=True)
            pltpu.make_async_copy(table.at[pl.ds(0, CH)], rows1, sem1).wait()

            @pl.when(j1 + 1 < nchunks)
            def _():
                pltpu.async_copy(table.at[sbuf.at[j1 + 1]], rows0, sem0)

            pltpu.sync_copy(rows1, acc.at[dbuf.at[j1]], add=True)
            return carry

        lax.fori_loop(0, nchunks // 2, body, 0)
        plsc.subcore_barrier()
        pltpu.sync_copy(acc.at[pl.ds(s * rpt, rpt)], out.at[c, pl.ds(s * rpt, rpt)])

    return mp_kernel


def _dense1(xp, W1, degp):
    NP, H = xp.shape[0], W1.shape[1]

    def body(x_ref, w_ref, degp_ref, hs_ref, h_ref, dinv_ref):
        h = jnp.dot(x_ref[...], w_ref[...], preferred_element_type=jnp.float32)
        deg = degp_ref[0] + degp_ref[1] + 1.0
        dinv = lax.rsqrt(deg)[:, None]
        h_ref[...] = h
        dinv_ref[...] = dinv
        hs_ref[...] = dinv * h

    return pl.pallas_call(
        body,
        out_shape=[
            jax.ShapeDtypeStruct((NP, H), jnp.float32),
            jax.ShapeDtypeStruct((NP, H), jnp.float32),
            jax.ShapeDtypeStruct((NP, 1), jnp.float32),
        ],
    )(xp, W1, degp)


def _dense2(s1p, dinv, h1, b1, W2, n_real):
    NP, H = h1.shape

    def body(sp_ref, dinv_ref, h_ref, b_ref, w_ref, hs_ref, h2_ref):
        dinv = dinv_ref[...]
        su = sp_ref[0] + sp_ref[1]
        out1 = jnp.maximum(dinv * su + dinv * dinv * h_ref[...] + b_ref[...][None, :], 0.0)
        h2 = jnp.dot(out1, w_ref[...], preferred_element_type=jnp.float32)
        mask = lax.broadcasted_iota(jnp.int32, (NP, 1), 0) < n_real
        h2_ref[...] = h2
        hs_ref[...] = jnp.where(mask, dinv * h2, 0.0)

    return pl.pallas_call(
        body,
        out_shape=[
            jax.ShapeDtypeStruct((NP, H), jnp.float32),
            jax.ShapeDtypeStruct((NP, H), jnp.float32),
        ],
    )(s1p, dinv, h1, b1, W2)


def _dense3(s2p, dinv, h2, b2, batch_p, W3, b3):
    NP, H = h2.shape

    def body(sp_ref, dinv_ref, h_ref, b_ref, batch_ref, w3_ref, b3_ref, out_ref):
        dinv = dinv_ref[...]
        su = sp_ref[0] + sp_ref[1]
        out2 = jnp.maximum(dinv * su + dinv * dinv * h_ref[...] + b_ref[...][None, :], 0.0)
        z = jnp.dot(out2, w3_ref[...], preferred_element_type=jnp.float32)
        gi = lax.broadcasted_iota(jnp.int32, (G, NP), 0)
        onehot = (batch_ref[...][None, :] == gi).astype(jnp.float32)
        sums = jnp.dot(onehot, z, preferred_element_type=jnp.float32)
        counts = jnp.sum(onehot, axis=1, keepdims=True)
        out_ref[...] = sums / jnp.maximum(counts, 1.0) + b3_ref[...][None, :]

    return pl.pallas_call(
        body,
        out_shape=jax.ShapeDtypeStruct((G, 1), jnp.float32),
    )(s2p, dinv, h2, b2, batch_p, W3, b3)


def kernel(x, edge_index, batch, W1, b1, W2, b2, W3, b3):
    N, F = x.shape
    H = W1.shape[1]
    E = edge_index.shape[1]
    NP = _pad_to(N + 1, NS * 128)
    EP = _pad_to(E, NW * CH * 2)

    src = edge_index[0].astype(jnp.int32)
    dst = edge_index[1].astype(jnp.int32)
    pad_e = jnp.full((EP - E,), N, jnp.int32)
    srcp = jnp.concatenate([src, pad_e]).reshape(EP // CH, CH)
    dstp = jnp.concatenate([dst, pad_e]).reshape(EP // CH, CH)
    xp = jnp.concatenate([x, jnp.zeros((NP - N, F), x.dtype)])
    batch_p = jnp.concatenate(
        [batch.astype(jnp.int32), jnp.full((NP - N,), G, jnp.int32)])
    zeros1 = jnp.zeros((NP,), jnp.float32)
    zeros2 = jnp.zeros((NP, H), jnp.float32)

    degp = _make_deg(EP, NP)(dstp, zeros1).reshape(NC, NP)
    hs1, h1, dinv = _dense1(xp, W1, degp)
    s1p = _make_mp(EP, NP, H)(hs1, srcp, dstp, zeros2)
    hs2, h2 = _dense2(s1p, dinv, h1, b1, W2, N)
    s2p = _make_mp(EP, NP, H)(hs2, srcp, dstp, zeros2)
    out = _dense3(s2p, dinv, h2, b2, batch_p, W3, b3)
    return out.reshape(-1)


# speedup vs baseline: 56.3416x; 2.1270x over previous
import functools

import jax
import jax.numpy as jnp
from jax import lax
from jax.experimental import pallas as pl
from jax.experimental.pallas import tpu as pltpu
from jax.experimental.pallas import tpu_sc as plsc

NC = 2
NS = 16
NW = NC * NS
CH = 128
G = 64


def _pad_to(n: int, m: int) -> int:
    return -(-n // m) * m


@functools.lru_cache(maxsize=None)
def _make_deg(EP: int, NP: int):
    nchunks = EP // (NW * CH)
    rpt = NP // NS
    mesh = plsc.VectorSubcoreMesh(core_axis_name="c", subcore_axis_name="s")

    @functools.partial(
        pl.kernel,
        out_type=jax.ShapeDtypeStruct((NC * NP,), jnp.float32),
        mesh=mesh,
        scratch_types=[
            pltpu.VMEM_SHARED((NP,), jnp.float32),
            pltpu.VMEM((nchunks, CH), jnp.int32),
            pltpu.VMEM((CH,), jnp.float32),
            pltpu.SemaphoreType.DMA,
        ],
    )
    def deg_kernel(dst2, zeros1, out, acc, dbuf, ones_v, ssem):
        c = lax.axis_index("c")
        s = lax.axis_index("s")
        tile = c * NS + s
        for i in range(CH // 16):
            ones_v[pl.ds(i * 16, 16)] = jnp.ones((16,), jnp.float32)
        pltpu.sync_copy(zeros1.at[pl.ds(s * rpt, rpt)], acc.at[pl.ds(s * rpt, rpt)])
        pltpu.sync_copy(dst2.at[pl.ds(tile * nchunks, nchunks)], dbuf)
        plsc.subcore_barrier()

        def body(j, carry):
            @pl.when(j >= 8)
            def _():
                pltpu.make_async_copy(
                    zeros1.at[pl.ds(0, CH)], ones_v, ssem).wait()
            pltpu.async_copy(ones_v, acc.at[dbuf.at[j]], ssem, add=True)
            return carry

        lax.fori_loop(0, nchunks, body, 0)
        for _ in range(min(nchunks, 8)):
            pltpu.make_async_copy(zeros1.at[pl.ds(0, CH)], ones_v, ssem).wait()
        plsc.subcore_barrier()
        pltpu.sync_copy(acc.at[pl.ds(s * rpt, rpt)],
                        out.at[pl.ds(c * NP + s * rpt, rpt)])

    return deg_kernel


@functools.lru_cache(maxsize=None)
def _make_mp(EP: int, NP: int, H: int):
    K = 8
    D = 4
    nchunks = EP // (NW * CH)
    assert nchunks % K == 0 and nchunks >= 2 * K
    rpt = NP // NS
    mesh = plsc.VectorSubcoreMesh(core_axis_name="c", subcore_axis_name="s")

    @functools.partial(
        pl.kernel,
        out_type=jax.ShapeDtypeStruct((NC, NP, H), jnp.float32),
        mesh=mesh,
        compiler_params=pltpu.CompilerParams(use_tc_tiling_on_sc=False),
        scratch_types=(
            [
                pltpu.VMEM_SHARED((NP, H), jnp.float32),
                pltpu.VMEM((nchunks, CH), jnp.int32),
                pltpu.VMEM((nchunks, CH), jnp.int32),
            ]
            + [pltpu.VMEM((CH, H), jnp.float32)] * K
            + [pltpu.SemaphoreType.DMA] * (2 * K)
        ),
    )
    def mp_kernel(table, src2, dst2, zeros2, out, acc, sbuf, dbuf, *rest):
        rows = rest[:K]
        gsem = rest[K:2 * K]
        ssem = rest[2 * K:3 * K]
        c = lax.axis_index("c")
        s = lax.axis_index("s")
        tile = c * NS + s
        pltpu.sync_copy(zeros2.at[pl.ds(s * rpt, rpt)], acc.at[pl.ds(s * rpt, rpt)])
        pltpu.sync_copy(src2.at[pl.ds(tile * nchunks, nchunks)], sbuf)
        pltpu.sync_copy(dst2.at[pl.ds(tile * nchunks, nchunks)], dbuf)
        plsc.subcore_barrier()
        for b in range(D):
            pltpu.async_copy(table.at[sbuf.at[b]], rows[b], gsem[b])

        def body(g, carry):
            for b in range(K):
                j = g * K + b
                bp = (b + D) % K
                pltpu.make_async_copy(table.at[pl.ds(0, CH)], rows[b], gsem[b]).wait()
                pltpu.async_copy(rows[b], acc.at[dbuf.at[j]], ssem[b], add=True)

                @pl.when(j + D < nchunks)
                def _():
                    @pl.when(j >= D)
                    def _():
                        pltpu.make_async_copy(
                            table.at[pl.ds(0, CH)], rows[bp], ssem[bp]).wait()
                    pltpu.async_copy(table.at[sbuf.at[j + D]], rows[bp], gsem[bp])

            return carry

        lax.fori_loop(0, nchunks // K, body, 0)
        for b in range(K):
            pltpu.make_async_copy(table.at[pl.ds(0, CH)], rows[b], ssem[b]).wait()
        plsc.subcore_barrier()
        pltpu.sync_copy(acc.at[pl.ds(s * rpt, rpt)], out.at[c, pl.ds(s * rpt, rpt)])

    return mp_kernel


def _dense1(xp, W1, degp):
    NP, H = xp.shape[0], W1.shape[1]

    def body(x_ref, w_ref, degp_ref, hs_ref, h_ref, dinv_ref):
        h = jnp.dot(x_ref[...], w_ref[...], preferred_element_type=jnp.float32)
        deg = degp_ref[0] + degp_ref[1] + 1.0
        dinv = lax.rsqrt(deg)[:, None]
        h_ref[...] = h
        dinv_ref[...] = dinv
        hs_ref[...] = dinv * h

    return pl.pallas_call(
        body,
        out_shape=[
            jax.ShapeDtypeStruct((NP, H), jnp.float32),
            jax.ShapeDtypeStruct((NP, H), jnp.float32),
            jax.ShapeDtypeStruct((NP, 1), jnp.float32),
        ],
    )(xp, W1, degp)


def _dense2(s1p, dinv, h1, b1, W2, n_real):
    NP, H = h1.shape

    def body(sp_ref, dinv_ref, h_ref, b_ref, w_ref, hs_ref, h2_ref):
        dinv = dinv_ref[...]
        su = sp_ref[0] + sp_ref[1]
        out1 = jnp.maximum(dinv * su + dinv * dinv * h_ref[...] + b_ref[...][None, :], 0.0)
        h2 = jnp.dot(out1, w_ref[...], preferred_element_type=jnp.float32)
        mask = lax.broadcasted_iota(jnp.int32, (NP, 1), 0) < n_real
        h2_ref[...] = h2
        hs_ref[...] = jnp.where(mask, dinv * h2, 0.0)

    return pl.pallas_call(
        body,
        out_shape=[
            jax.ShapeDtypeStruct((NP, H), jnp.float32),
            jax.ShapeDtypeStruct((NP, H), jnp.float32),
        ],
    )(s1p, dinv, h1, b1, W2)


def _dense3(s2p, dinv, h2, b2, batch_p, W3, b3):
    NP, H = h2.shape

    def body(sp_ref, dinv_ref, h_ref, b_ref, batch_ref, w3_ref, b3_ref, out_ref):
        dinv = dinv_ref[...]
        su = sp_ref[0] + sp_ref[1]
        out2 = jnp.maximum(dinv * su + dinv * dinv * h_ref[...] + b_ref[...][None, :], 0.0)
        z = jnp.dot(out2, w3_ref[...], preferred_element_type=jnp.float32)
        gi = lax.broadcasted_iota(jnp.int32, (G, NP), 0)
        onehot = (batch_ref[...][None, :] == gi).astype(jnp.float32)
        sums = jnp.dot(onehot, z, preferred_element_type=jnp.float32)
        counts = jnp.sum(onehot, axis=1, keepdims=True)
        out_ref[...] = sums / jnp.maximum(counts, 1.0) + b3_ref[...][None, :]

    return pl.pallas_call(
        body,
        out_shape=jax.ShapeDtypeStruct((G, 1), jnp.float32),
    )(s2p, dinv, h2, b2, batch_p, W3, b3)


def kernel(x, edge_index, batch, W1, b1, W2, b2, W3, b3):
    N, F = x.shape
    H = W1.shape[1]
    E = edge_index.shape[1]
    NP = _pad_to(N + 1, NS * 128)
    EP = _pad_to(E, NW * CH * 8)

    src = edge_index[0].astype(jnp.int32)
    dst = edge_index[1].astype(jnp.int32)
    pad_e = N + jnp.arange(EP - E, dtype=jnp.int32) % (NP - N)
    srcp = jnp.concatenate([src, pad_e]).reshape(EP // CH, CH)
    dstp = jnp.concatenate([dst, pad_e]).reshape(EP // CH, CH)
    xp = jnp.concatenate([x, jnp.zeros((NP - N, F), x.dtype)])
    batch_p = jnp.concatenate(
        [batch.astype(jnp.int32), jnp.full((NP - N,), G, jnp.int32)])
    zeros1 = jnp.zeros((NP,), jnp.float32)
    zeros2 = jnp.zeros((NP, H), jnp.float32)

    degp = _make_deg(EP, NP)(dstp, zeros1).reshape(NC, NP)
    hs1, h1, dinv = _dense1(xp, W1, degp)
    s1p = _make_mp(EP, NP, H)(hs1, srcp, dstp, zeros2)
    hs2, h2 = _dense2(s1p, dinv, h1, b1, W2, N)
    s2p = _make_mp(EP, NP, H)(hs2, srcp, dstp, zeros2)
    out = _dense3(s2p, dinv, h2, b2, batch_p, W3, b3)
    return out.reshape(-1)


# speedup vs baseline: 56.5643x; 1.0040x over previous
import functools

import jax
import jax.numpy as jnp
from jax import lax
from jax.experimental import pallas as pl
from jax.experimental.pallas import tpu as pltpu
from jax.experimental.pallas import tpu_sc as plsc

NC = 2
NS = 16
NW = NC * NS
CH = 128
G = 64


def _pad_to(n: int, m: int) -> int:
    return -(-n // m) * m


@functools.lru_cache(maxsize=None)
def _make_deg(EP: int, NP: int):
    nchunks = EP // (NW * CH)
    rpt = NP // NS
    mesh = plsc.VectorSubcoreMesh(core_axis_name="c", subcore_axis_name="s")

    @functools.partial(
        pl.kernel,
        out_type=jax.ShapeDtypeStruct((NC * NP,), jnp.float32),
        mesh=mesh,
        scratch_types=[
            pltpu.VMEM_SHARED((NP,), jnp.float32),
            pltpu.VMEM((nchunks, CH), jnp.int32),
            pltpu.VMEM((CH,), jnp.float32),
            pltpu.SemaphoreType.DMA,
        ],
    )
    def deg_kernel(dst2, zeros1, out, acc, dbuf, ones_v, ssem):
        c = lax.axis_index("c")
        s = lax.axis_index("s")
        tile = c * NS + s
        for i in range(CH // 16):
            ones_v[pl.ds(i * 16, 16)] = jnp.ones((16,), jnp.float32)
        pltpu.sync_copy(zeros1.at[pl.ds(s * rpt, rpt)], acc.at[pl.ds(s * rpt, rpt)])
        pltpu.sync_copy(dst2.at[pl.ds(tile * nchunks, nchunks)], dbuf)
        plsc.subcore_barrier()

        def body(j, carry):
            @pl.when(j >= 8)
            def _():
                pltpu.make_async_copy(
                    zeros1.at[pl.ds(0, CH)], ones_v, ssem).wait()
            pltpu.async_copy(ones_v, acc.at[dbuf.at[j]], ssem, add=True)
            return carry

        lax.fori_loop(0, nchunks, body, 0)
        for _ in range(min(nchunks, 8)):
            pltpu.make_async_copy(zeros1.at[pl.ds(0, CH)], ones_v, ssem).wait()
        plsc.subcore_barrier()
        pltpu.sync_copy(acc.at[pl.ds(s * rpt, rpt)],
                        out.at[pl.ds(c * NP + s * rpt, rpt)])

    return deg_kernel


@functools.lru_cache(maxsize=None)
def _make_mp(EP: int, NP: int, H: int):
    K = 8
    D = 4
    nchunks = EP // (NW * CH)
    assert nchunks % K == 0 and nchunks >= 2 * K
    rpt = NP // NS
    mesh = plsc.VectorSubcoreMesh(core_axis_name="c", subcore_axis_name="s")

    @functools.partial(
        pl.kernel,
        out_type=jax.ShapeDtypeStruct((NC, NP, H), jnp.float32),
        mesh=mesh,
        compiler_params=pltpu.CompilerParams(use_tc_tiling_on_sc=False),
        scratch_types=(
            [
                pltpu.VMEM_SHARED((NP, H), jnp.float32),
                pltpu.VMEM((nchunks, CH), jnp.int32),
                pltpu.VMEM((nchunks, CH), jnp.int32),
            ]
            + [pltpu.VMEM((CH, H), jnp.float32)] * K
            + [pltpu.SemaphoreType.DMA] * (2 * K)
        ),
    )
    def mp_kernel(table, src2, dst2, zeros2, out, acc, sbuf, dbuf, *rest):
        rows = rest[:K]
        gsem = rest[K:2 * K]
        ssem = rest[2 * K:3 * K]
        c = lax.axis_index("c")
        s = lax.axis_index("s")
        tile = c * NS + s
        pltpu.sync_copy(zeros2.at[pl.ds(s * rpt, rpt)], acc.at[pl.ds(s * rpt, rpt)])
        pltpu.sync_copy(src2.at[pl.ds(tile * nchunks, nchunks)], sbuf)
        pltpu.sync_copy(dst2.at[pl.ds(tile * nchunks, nchunks)], dbuf)
        plsc.subcore_barrier()
        for b in range(D):
            pltpu.async_copy(table.at[sbuf.at[b]], rows[b], gsem[b])

        def body(g, carry):
            for b in range(K):
                j = g * K + b
                bp = (b + D) % K
                pltpu.make_async_copy(table.at[pl.ds(0, CH)], rows[b], gsem[b]).wait()
                pltpu.async_copy(rows[b], acc.at[dbuf.at[j]], ssem[b], add=True)

                @pl.when(j + D < nchunks)
                def _():
                    @pl.when(j >= D)
                    def _():
                        pltpu.make_async_copy(
                            table.at[pl.ds(0, CH)], rows[bp], ssem[bp]).wait()
                    pltpu.async_copy(table.at[sbuf.at[j + D]], rows[bp], gsem[bp])

            return carry

        lax.fori_loop(0, nchunks // K, body, 0)
        for b in range(K):
            pltpu.make_async_copy(table.at[pl.ds(0, CH)], rows[b], ssem[b]).wait()
        plsc.subcore_barrier()
        pltpu.sync_copy(acc.at[pl.ds(s * rpt, rpt)], out.at[c, pl.ds(s * rpt, rpt)])

    return mp_kernel


def _dense1(x, W1, degp, NP):
    N, H = x.shape[0], W1.shape[1]

    def body(x_ref, w_ref, degp_ref, hs_ref, h_ref, dinv_ref):
        h = jnp.dot(x_ref[...], w_ref[...], preferred_element_type=jnp.float32)
        deg = degp_ref[0] + degp_ref[1] + 1.0
        dinv = lax.rsqrt(deg)[:, None]
        zpad = jnp.zeros((NP - N, H), jnp.float32)
        h_ref[pl.ds(0, N)] = h
        h_ref[pl.ds(N, NP - N)] = zpad
        dinv_ref[...] = dinv
        hs_ref[pl.ds(0, N)] = dinv[:N] * h
        hs_ref[pl.ds(N, NP - N)] = zpad

    return pl.pallas_call(
        body,
        out_shape=[
            jax.ShapeDtypeStruct((NP, H), jnp.float32),
            jax.ShapeDtypeStruct((NP, H), jnp.float32),
            jax.ShapeDtypeStruct((NP, 1), jnp.float32),
        ],
    )(x, W1, degp)


def _dense2(s1p, dinv, h1, b1, W2, n_real):
    NP, H = h1.shape

    def body(sp_ref, dinv_ref, h_ref, b_ref, w_ref, hs_ref, h2_ref):
        dinv = dinv_ref[...]
        su = sp_ref[0] + sp_ref[1]
        out1 = jnp.maximum(dinv * su + dinv * dinv * h_ref[...] + b_ref[...][None, :], 0.0)
        h2 = jnp.dot(out1, w_ref[...], preferred_element_type=jnp.float32)
        mask = lax.broadcasted_iota(jnp.int32, (NP, 1), 0) < n_real
        h2_ref[...] = h2
        hs_ref[...] = jnp.where(mask, dinv * h2, 0.0)

    return pl.pallas_call(
        body,
        out_shape=[
            jax.ShapeDtypeStruct((NP, H), jnp.float32),
            jax.ShapeDtypeStruct((NP, H), jnp.float32),
        ],
    )(s1p, dinv, h1, b1, W2)


def _dense3(s2p, dinv, h2, b2, batch_p, W3, b3):
    NP, H = h2.shape

    def body(sp_ref, dinv_ref, h_ref, b_ref, batch_ref, w3_ref, b3_ref, out_ref):
        dinv = dinv_ref[...]
        su = sp_ref[0] + sp_ref[1]
        out2 = jnp.maximum(dinv * su + dinv * dinv * h_ref[...] + b_ref[...][None, :], 0.0)
        z = jnp.dot(out2, w3_ref[...], preferred_element_type=jnp.float32)
        gi = lax.broadcasted_iota(jnp.int32, (G, NP), 0)
        onehot = (batch_ref[...][None, :] == gi).astype(jnp.float32)
        sums = jnp.dot(onehot, z, preferred_element_type=jnp.float32)
        counts = jnp.sum(onehot, axis=1, keepdims=True)
        out_ref[...] = sums / jnp.maximum(counts, 1.0) + b3_ref[...][None, :]

    return pl.pallas_call(
        body,
        out_shape=jax.ShapeDtypeStruct((G, 1), jnp.float32),
    )(s2p, dinv, h2, b2, batch_p, W3, b3)


def kernel(x, edge_index, batch, W1, b1, W2, b2, W3, b3):
    N, F = x.shape
    H = W1.shape[1]
    E = edge_index.shape[1]
    NP = _pad_to(N + 1, NS * 128)
    EP = _pad_to(E, NW * CH * 8)

    src = edge_index[0].astype(jnp.int32)
    dst = edge_index[1].astype(jnp.int32)
    pad_e = N + jnp.arange(EP - E, dtype=jnp.int32) % (NP - N)
    srcp = jnp.concatenate([src, pad_e]).reshape(EP // CH, CH)
    dstp = jnp.concatenate([dst, pad_e]).reshape(EP // CH, CH)
    batch_p = jnp.concatenate(
        [batch.astype(jnp.int32), jnp.full((NP - N,), G, jnp.int32)])
    zeros1 = jnp.zeros((NP,), jnp.float32)
    zeros2 = jnp.zeros((NP, H), jnp.float32)

    degp = _make_deg(EP, NP)(dstp, zeros1).reshape(NC, NP)
    hs1, h1, dinv = _dense1(x, W1, degp, NP)
    s1p = _make_mp(EP, NP, H)(hs1, srcp, dstp, zeros2)
    hs2, h2 = _dense2(s1p, dinv, h1, b1, W2, N)
    s2p = _make_mp(EP, NP, H)(hs2, srcp, dstp, zeros2)
    out = _dense3(s2p, dinv, h2, b2, batch_p, W3, b3)
    return out.reshape(-1)


# speedup vs baseline: 58.6054x; 1.0361x over previous
import functools

import jax
import jax.numpy as jnp
from jax import lax
from jax.experimental import pallas as pl
from jax.experimental.pallas import tpu as pltpu
from jax.experimental.pallas import tpu_sc as plsc

NC = 2
NS = 16
NW = NC * NS
CH = 128
G = 64


def _pad_to(n: int, m: int) -> int:
    return -(-n // m) * m


@functools.lru_cache(maxsize=None)
def _make_deg(EP: int, NP: int):
    nchunks = EP // (NW * CH)
    rpt = NP // NS
    mesh = plsc.VectorSubcoreMesh(core_axis_name="c", subcore_axis_name="s")

    @functools.partial(
        pl.kernel,
        out_type=jax.ShapeDtypeStruct((NC * NP,), jnp.float32),
        mesh=mesh,
        scratch_types=[
            pltpu.VMEM_SHARED((NP,), jnp.float32),
            pltpu.VMEM((nchunks, CH), jnp.int32),
            pltpu.VMEM((CH,), jnp.float32),
            pltpu.SemaphoreType.DMA,
        ],
    )
    def deg_kernel(dst2, zeros1, out, acc, dbuf, ones_v, ssem):
        c = lax.axis_index("c")
        s = lax.axis_index("s")
        tile = c * NS + s
        for i in range(CH // 16):
            ones_v[pl.ds(i * 16, 16)] = jnp.ones((16,), jnp.float32)
        pltpu.sync_copy(zeros1.at[pl.ds(s * rpt, rpt)], acc.at[pl.ds(s * rpt, rpt)])
        pltpu.sync_copy(dst2.at[pl.ds(tile * nchunks, nchunks)], dbuf)
        plsc.subcore_barrier()

        def body(j, carry):
            @pl.when(j >= 8)
            def _():
                pltpu.make_async_copy(
                    zeros1.at[pl.ds(0, CH)], ones_v, ssem).wait()
            pltpu.async_copy(ones_v, acc.at[dbuf.at[j]], ssem, add=True)
            return carry

        lax.fori_loop(0, nchunks, body, 0)
        for _ in range(min(nchunks, 8)):
            pltpu.make_async_copy(zeros1.at[pl.ds(0, CH)], ones_v, ssem).wait()
        plsc.subcore_barrier()
        pltpu.sync_copy(acc.at[pl.ds(s * rpt, rpt)],
                        out.at[pl.ds(c * NP + s * rpt, rpt)])

    return deg_kernel


@functools.lru_cache(maxsize=None)
def _make_mp(EP: int, NP: int, H: int):
    K = 8
    D = 4
    nchunks = EP // (NW * CH)
    assert nchunks % K == 0 and nchunks >= 2 * K
    rpt = NP // NS
    mesh = plsc.VectorSubcoreMesh(core_axis_name="c", subcore_axis_name="s")

    @functools.partial(
        pl.kernel,
        out_type=jax.ShapeDtypeStruct((NC, NP, H), jnp.float32),
        mesh=mesh,
        compiler_params=pltpu.CompilerParams(use_tc_tiling_on_sc=False),
        scratch_types=(
            [
                pltpu.VMEM_SHARED((NP, H), jnp.float32),
                pltpu.VMEM((nchunks, CH), jnp.int32),
                pltpu.VMEM((nchunks, CH), jnp.int32),
            ]
            + [pltpu.VMEM((CH, H), jnp.float32)] * K
            + [pltpu.SemaphoreType.DMA] * (2 * K)
        ),
    )
    def mp_kernel(table, src2, dst2, zeros2, out, acc, sbuf, dbuf, *rest):
        rows = rest[:K]
        gsem = rest[K:2 * K]
        ssem = rest[2 * K:3 * K]
        c = lax.axis_index("c")
        s = lax.axis_index("s")
        tile = c * NS + s
        pltpu.sync_copy(zeros2.at[pl.ds(s * rpt, rpt)], acc.at[pl.ds(s * rpt, rpt)])
        pltpu.sync_copy(src2.at[pl.ds(tile * nchunks, nchunks)], sbuf)
        pltpu.sync_copy(dst2.at[pl.ds(tile * nchunks, nchunks)], dbuf)
        plsc.subcore_barrier()
        for b in range(D):
            pltpu.async_copy(table.at[sbuf.at[b]], rows[b], gsem[b])

        def body(g, carry):
            for b in range(K):
                j = g * K + b
                bp = (b + D) % K
                pltpu.make_async_copy(table.at[pl.ds(0, CH)], rows[b], gsem[b]).wait()
                pltpu.async_copy(rows[b], acc.at[dbuf.at[j]], ssem[b], add=True)

                @pl.when(j + D < nchunks)
                def _():
                    @pl.when(j >= D)
                    def _():
                        pltpu.make_async_copy(
                            table.at[pl.ds(0, CH)], rows[bp], ssem[bp]).wait()
                    pltpu.async_copy(table.at[sbuf.at[j + D]], rows[bp], gsem[bp])

            return carry

        lax.fori_loop(0, nchunks // K, body, 0)
        for b in range(K):
            pltpu.make_async_copy(table.at[pl.ds(0, CH)], rows[b], ssem[b]).wait()
        plsc.subcore_barrier()
        pltpu.sync_copy(acc.at[pl.ds(s * rpt, rpt)], out.at[c, pl.ds(s * rpt, rpt)])

    return mp_kernel


def _dense1(x, W1, degp, NP):
    N, H = x.shape[0], W1.shape[1]

    def body(x_ref, w_ref, degp_ref, hs_ref, dinv_ref):
        h = jnp.dot(x_ref[...], w_ref[...], preferred_element_type=jnp.float32)
        deg = degp_ref[0] + degp_ref[1] + 1.0
        dinv = lax.rsqrt(deg)[:, None]
        dinv_ref[...] = dinv
        hs_ref[pl.ds(0, N)] = dinv[:N] * h
        hs_ref[pl.ds(N, NP - N)] = jnp.zeros((NP - N, H), jnp.float32)

    return pl.pallas_call(
        body,
        out_shape=[
            jax.ShapeDtypeStruct((NP, H), jnp.float32),
            jax.ShapeDtypeStruct((NP, 1), jnp.float32),
        ],
    )(x, W1, degp)


def _dense2(s1p, dinv, hs1, b1, W2):
    NP, H = hs1.shape

    def body(sp_ref, dinv_ref, hs1_ref, b_ref, w_ref, hs_ref):
        dinv = dinv_ref[...]
        su = sp_ref[0] + sp_ref[1]
        out1 = jnp.maximum(
            dinv * su + dinv * hs1_ref[...] + b_ref[...][None, :], 0.0)
        hs_ref[...] = dinv * jnp.dot(
            out1, w_ref[...], preferred_element_type=jnp.float32)

    return pl.pallas_call(
        body,
        out_shape=jax.ShapeDtypeStruct((NP, H), jnp.float32),
    )(s1p, dinv, hs1, b1, W2)


def _dense3(s2p, dinv, hs2, b2, batch_p, W3, b3):
    NP, H = hs2.shape

    def body(sp_ref, dinv_ref, hs2_ref, b_ref, batch_ref, w3_ref, b3_ref, out_ref):
        dinv = dinv_ref[...]
        su = sp_ref[0] + sp_ref[1]
        out2 = jnp.maximum(
            dinv * su + dinv * hs2_ref[...] + b_ref[...][None, :], 0.0)
        z = jnp.dot(out2, w3_ref[...], preferred_element_type=jnp.float32)
        gi = lax.broadcasted_iota(jnp.int32, (G, NP), 0)
        onehot = (batch_ref[...][None, :] == gi).astype(jnp.float32)
        sums = jnp.dot(onehot, z, preferred_element_type=jnp.float32)
        counts = jnp.sum(onehot, axis=1, keepdims=True)
        out_ref[...] = sums / jnp.maximum(counts, 1.0) + b3_ref[...][None, :]

    return pl.pallas_call(
        body,
        out_shape=jax.ShapeDtypeStruct((G, 1), jnp.float32),
    )(s2p, dinv, hs2, b2, batch_p, W3, b3)


def kernel(x, edge_index, batch, W1, b1, W2, b2, W3, b3):
    N, F = x.shape
    H = W1.shape[1]
    E = edge_index.shape[1]
    NP = _pad_to(N + 1, NS * 128)
    EP = _pad_to(E, NW * CH * 8)

    src = edge_index[0].astype(jnp.int32)
    dst = edge_index[1].astype(jnp.int32)
    pad_e = N + jnp.arange(EP - E, dtype=jnp.int32) % (NP - N)
    srcp = jnp.concatenate([src, pad_e]).reshape(EP // CH, CH)
    dstp = jnp.concatenate([dst, pad_e]).reshape(EP // CH, CH)
    batch_p = jnp.concatenate(
        [batch.astype(jnp.int32), jnp.full((NP - N,), G, jnp.int32)])
    zeros1 = jnp.zeros((NP,), jnp.float32)
    zeros2 = jnp.zeros((NP, H), jnp.float32)

    degp = _make_deg(EP, NP)(dstp, zeros1).reshape(NC, NP)
    hs1, dinv = _dense1(x, W1, degp, NP)
    s1p = _make_mp(EP, NP, H)(hs1, srcp, dstp, zeros2)
    hs2 = _dense2(s1p, dinv, hs1, b1, W2)
    s2p = _make_mp(EP, NP, H)(hs2, srcp, dstp, zeros2)
    out = _dense3(s2p, dinv, hs2, b2, batch_p, W3, b3)
    return out.reshape(-1)
